# baseline (speedup 1.0000x reference)
import functools
from functools import partial

import jax
import jax.numpy as jnp
from jax import lax
from jax.experimental import pallas as pl
from jax.experimental.pallas import tpu as pltpu
from jax.experimental.pallas import tpu_sc as plsc

GDEP = 2
ALPHA = 0.05
KS = [0.9, 0.8, 0.7, 0.6, 0.5]

_HI = jax.lax.Precision.HIGHEST
BT = 128


PADU = 256


def _pad_to(v):
    return ((v + PADU - 1) // PADU) * PADU


def _transpose_body(i_ref, o_ref):
    o_ref[...] = jnp.swapaxes(i_ref[...], 0, 1)


def _transpose(g):
    n = g.shape[0]
    gr = n // BT
    return pl.pallas_call(
        _transpose_body,
        grid=(gr, gr),
        in_specs=[pl.BlockSpec((BT, BT), lambda i, j: (i, j))],
        out_specs=pl.BlockSpec((BT, BT), lambda i, j: (j, i)),
        out_shape=jax.ShapeDtypeStruct((n, n), g.dtype),
    )(g)


def _normadj_body(g_ref, gt_ref, dr_ref, dc_ref, a_ref, at_ref):
    i = pl.program_id(0)
    j = pl.program_id(1)
    rows = i * BT + jax.lax.broadcasted_iota(jnp.int32, (BT, BT), 0)
    cols = j * BT + jax.lax.broadcasted_iota(jnp.int32, (BT, BT), 1)
    eye = (rows == cols).astype(jnp.float32)
    a_ref[...] = (g_ref[...] + eye) / dr_ref[...]
    at_ref[...] = (gt_ref[...] + eye) / dc_ref[...]


def _normadj(g, gt, dr, dc):
    n = g.shape[0]
    spec = pl.BlockSpec((BT, BT), lambda i, j: (i, j))
    dspec = pl.BlockSpec((BT, 1), lambda i, j: (i, 0))
    return pl.pallas_call(
        _normadj_body,
        grid=(n // BT, n // BT),
        in_specs=[spec, spec, dspec, dspec],
        out_specs=[spec, spec],
        out_shape=[jax.ShapeDtypeStruct((n, n), jnp.float32),
                   jax.ShapeDtypeStruct((n, n), jnp.float32)],
    )(g, gt, dr, dc)


def _prop_body(a_ref, xk_ref, x0_ref, o_ref):
    acc = jnp.dot(a_ref[...].astype(jnp.bfloat16),
                  xk_ref[...].astype(jnp.bfloat16),
                  preferred_element_type=jnp.float32)
    o_ref[...] = ALPHA * x0_ref[...] + (1.0 - ALPHA) * acc


def _propagate(a, x, x0):
    n = a.shape[0]
    f = x.shape[1]
    return pl.pallas_call(
        _prop_body,
        grid=(n // BT, f // BT),
        in_specs=[
            pl.BlockSpec((BT, n), lambda m, c: (m, 0)),
            pl.BlockSpec((n, BT), lambda m, c: (0, c)),
            pl.BlockSpec((BT, BT), lambda m, c: (m, c)),
        ],
        out_specs=pl.BlockSpec((BT, BT), lambda m, c: (m, c)),
        out_shape=jax.ShapeDtypeStruct((n, f), jnp.float32),
    )(a, x, x0)


def _mix_body(x0, x1, x2, x3, x4, k1_ref, k2_ref, b1_ref, b2_ref, pw_ref,
              pb_ref, c_ref, s_ref, *, nc):
    c = pl.program_id(1)
    cat1 = jnp.concatenate([x0[...], x1[...], x2[...]], axis=1)
    cat2 = jnp.concatenate([x0[...], x3[...], x4[...]], axis=1)
    m1 = jnp.dot(cat1.astype(jnp.bfloat16), k1_ref[...].astype(jnp.bfloat16),
                 preferred_element_type=jnp.float32)
    m2 = jnp.dot(cat2.astype(jnp.bfloat16), k2_ref[...].astype(jnp.bfloat16),
                 preferred_element_type=jnp.float32)
    cb = jnp.maximum((m1 + b1_ref[...]) + (m2 + b2_ref[...]), 0.0)
    c_ref[...] = cb
    part = jnp.dot(cb.astype(jnp.bfloat16), pw_ref[...].astype(jnp.bfloat16),
                   preferred_element_type=jnp.float32)
    tot = jnp.where(c == 0, part, s_ref[...] + part)
    s_ref[...] = jnp.where(c == nc - 1,
                           jax.nn.sigmoid(tot + pb_ref[...]), tot)


def _mix_scores(xs, ks, biases, pw, pb):
    n, f = xs[0].shape
    nc = f // BT
    xspec = pl.BlockSpec((BT, BT), lambda m, c: (m, c))
    kspec = pl.BlockSpec((3 * BT, BT), lambda m, c: (0, 0))
    bspec = pl.BlockSpec((1, BT), lambda m, c: (0, c))
    return pl.pallas_call(
        partial(_mix_body, nc=nc),
        grid=(n // BT, nc),
        in_specs=[xspec] * 5 + [kspec] * 2 + [bspec] * 2 + [
            pl.BlockSpec((BT, 1), lambda m, c: (c, 0)),
            pl.BlockSpec((1, 1), lambda m, c: (0, 0)),
        ],
        out_specs=[
            pl.BlockSpec((BT, BT), lambda m, c: (m, c)),
            pl.BlockSpec((BT, 1), lambda m, c: (m, 0)),
        ],
        out_shape=[
            jax.ShapeDtypeStruct((n, f), jnp.float32),
            jax.ShapeDtypeStruct((n, 1), jnp.float32),
        ],
    )(*xs, *ks, *biases, pw, pb)


def _rank_body(s_ref, sall_ref, r_ref, *, n_real, npad):
    i0 = pl.program_id(0) * BT
    si = s_ref[...]
    sj = sall_ref[...]
    rows = i0 + jax.lax.broadcasted_iota(jnp.int32, (BT, npad), 0)
    cols = jax.lax.broadcasted_iota(jnp.int32, (BT, npad), 1)
    vj = cols < n_real
    gt = (sj > si) & vj
    eq = (sj == si) & (cols < rows) & vj
    rank = jnp.sum(gt.astype(jnp.int32) + eq.astype(jnp.int32),
                   axis=1, keepdims=True)
    rank = jnp.where(rows[:, :1] < n_real, rank, jnp.int32(2**30))
    r_ref[...] = rank


def _rank(s, srow, n_real):
    npad = s.shape[0]
    return pl.pallas_call(
        partial(_rank_body, n_real=n_real, npad=npad),
        grid=(npad // BT,),
        in_specs=[
            pl.BlockSpec((BT, 1), lambda i: (i, 0)),
            pl.BlockSpec((1, npad), lambda i: (0, 0)),
        ],
        out_specs=pl.BlockSpec((BT, 1), lambda i: (i, 0)),
        out_shape=jax.ShapeDtypeStruct((npad, 1), jnp.int32),
    )(s, srow)


def _select_body(rrow_ref, srow_ref, i_ref, v_ref, m_ref, *, kk_real, npad):
    r0 = pl.program_id(0) * BT
    rglob = r0 + jax.lax.broadcasted_iota(jnp.int32, (BT, npad), 0)
    cols = jax.lax.broadcasted_iota(jnp.int32, (BT, npad), 1)
    onehot = (rrow_ref[...] == rglob) & (rglob < kk_real)
    valid = rglob[:, :1] < kk_real
    idx = jnp.sum(jnp.where(onehot, cols, 0), axis=1, keepdims=True)
    val = jnp.sum(jnp.where(onehot, srow_ref[...], 0.0), axis=1, keepdims=True)
    i_ref[...] = jnp.where(valid, idx, 0)
    v_ref[...] = jnp.where(valid, val, 0.0)
    m_ref[...] = onehot.astype(jnp.float32)


def _select(rrow, srow, kk_real, kkp):
    npad = rrow.shape[1]
    return pl.pallas_call(
        partial(_select_body, kk_real=kk_real, npad=npad),
        grid=(kkp // BT,),
        in_specs=[
            pl.BlockSpec((1, npad), lambda i: (0, 0)),
            pl.BlockSpec((1, npad), lambda i: (0, 0)),
        ],
        out_specs=[
            pl.BlockSpec((BT, 1), lambda i: (i, 0)),
            pl.BlockSpec((BT, 1), lambda i: (i, 0)),
            pl.BlockSpec((BT, npad), lambda i: (i, 0)),
        ],
        out_shape=[
            jax.ShapeDtypeStruct((kkp, 1), jnp.int32),
            jax.ShapeDtypeStruct((kkp, 1), jnp.float32),
            jax.ShapeDtypeStruct((kkp, npad), jnp.float32),
        ],
    )(rrow, srow)


def _sc_gather(idx, tables):
    kkp = idx.shape[0]
    NC, NS = 2, 16
    bpw = kkp // (NC * NS)
    CH = 8
    nt = len(tables)
    mesh = plsc.VectorSubcoreMesh(core_axis_name="c", subcore_axis_name="s")
    outs = [jax.ShapeDtypeStruct((kkp, t.shape[1]), t.dtype) for t in tables]
    scr = ([pltpu.VMEM((CH,), jnp.int32)]
           + [pltpu.VMEM((CH, t.shape[1]), t.dtype) for t in tables]
           + [pltpu.SemaphoreType.DMA])

    @functools.partial(pl.kernel, mesh=mesh, out_type=outs, scratch_types=scr)
    def k(*refs):
        idx_hbm = refs[0]
        tab = refs[1:1 + nt]
        out = refs[1 + nt:1 + 2 * nt]
        idx_v = refs[1 + 2 * nt]
        bufs = refs[2 + 2 * nt:2 + 3 * nt]
        sem = refs[2 + 3 * nt]
        wid = lax.axis_index("s") * NC + lax.axis_index("c")
        base = wid * bpw
        for ch in range(bpw // CH):
            off = base + ch * CH
            pltpu.sync_copy(idx_hbm.at[pl.ds(off, CH)], idx_v)
            for t in range(nt):
                pltpu.async_copy(tab[t].at[idx_v], bufs[t], sem).wait()
                pltpu.sync_copy(bufs[t], out[t].at[pl.ds(off, CH)])

    res = k(idx, *tables)
    return tuple(res) if isinstance(res, (list, tuple)) else (res,)


def _scale_body(c_ref, v_ref, o_ref):
    o_ref[...] = c_ref[...] * v_ref[...]


def _scale(csel, vals):
    kkp, f = csel.shape
    return pl.pallas_call(
        _scale_body,
        grid=(kkp // BT,),
        in_specs=[pl.BlockSpec((BT, f), lambda i: (i, 0)),
                  pl.BlockSpec((BT, 1), lambda i: (i, 0))],
        out_specs=pl.BlockSpec((BT, f), lambda i: (i, 0)),
        out_shape=jax.ShapeDtypeStruct((kkp, f), jnp.float32),
    )(csel, vals)


def _binmm_body(a_ref, b_ref, un_ref, deg_ref, acc_ref, *, nk, kk_real):
    k = pl.program_id(2)
    j = pl.program_id(1)
    m = pl.program_id(0)

    @pl.when(k == 0)
    def _():
        acc_ref[...] = jnp.zeros_like(acc_ref)

    arows = m * BT + jax.lax.broadcasted_iota(jnp.int32, (BT, BT), 0)
    brows = j * BT + jax.lax.broadcasted_iota(jnp.int32, (BT, BT), 0)
    a = ((a_ref[...] != 0) & (arows < kk_real)).astype(jnp.bfloat16)
    b = ((b_ref[...] != 0) & (brows < kk_real)).astype(jnp.bfloat16)
    acc_ref[...] += jax.lax.dot_general(
        a, b, (((1,), (1,)), ((), ())),
        preferred_element_type=jnp.float32)

    @pl.when(k == nk - 1)
    def _():
        un = (acc_ref[...] != 0).astype(jnp.float32)
        un_ref[...] = un
        part = jnp.sum(un, axis=1, keepdims=True)
        deg_ref[...] = jnp.where(j == 0, part, deg_ref[...] + part)


def _binmm(gsel, gtsel, kk_real):
    kkp, npad = gsel.shape
    nk = npad // BT
    return pl.pallas_call(
        partial(_binmm_body, nk=nk, kk_real=kk_real),
        grid=(kkp // BT, kkp // BT, nk),
        in_specs=[
            pl.BlockSpec((BT, BT), lambda m, j, k: (m, k)),
            pl.BlockSpec((BT, BT), lambda m, j, k: (j, k)),
        ],
        out_specs=[
            pl.BlockSpec((BT, BT), lambda m, j, k: (m, j)),
            pl.BlockSpec((BT, 1), lambda m, j, k: (m, 0)),
        ],
        out_shape=[
            jax.ShapeDtypeStruct((kkp, kkp), jnp.float32),
            jax.ShapeDtypeStruct((kkp, 1), jnp.float32),
        ],
        scratch_shapes=[pltpu.VMEM((BT, BT), jnp.float32)],
    )(gsel, gtsel)


def _rownorm_body(un_ref, deg_ref, o_ref):
    d = deg_ref[...]
    o_ref[...] = jnp.where(d > 0, un_ref[...] / d, 0.0)


def _rownorm(un, deg):
    kkp = un.shape[0]
    return pl.pallas_call(
        _rownorm_body,
        grid=(kkp // BT,),
        in_specs=[
            pl.BlockSpec((BT, kkp), lambda i: (i, 0)),
            pl.BlockSpec((BT, 1), lambda i: (i, 0)),
        ],
        out_specs=pl.BlockSpec((BT, kkp), lambda i: (i, 0)),
        out_shape=jax.ShapeDtypeStruct((kkp, kkp), jnp.float32),
    )(un, deg)


def _unpool_body(m_ref, p_ref, base_ref, o_ref):
    acc = jax.lax.dot_general(
        m_ref[...], p_ref[...], (((0,), (0,)), ((), ())),
        preferred_element_type=jnp.float32,
        precision=jax.lax.Precision.HIGHEST)
    o_ref[...] = base_ref[...] + acc


def _unpool_add(base, p, M):
    nprev, f = base.shape
    kkp = M.shape[0]
    return pl.pallas_call(
        _unpool_body,
        grid=(nprev // BT, f // BT),
        in_specs=[
            pl.BlockSpec((kkp, BT), lambda m, c: (0, m)),
            pl.BlockSpec((kkp, BT), lambda m, c: (0, c)),
            pl.BlockSpec((BT, BT), lambda m, c: (m, c)),
        ],
        out_specs=pl.BlockSpec((BT, BT), lambda m, c: (m, c)),
        out_shape=jax.ShapeDtypeStruct((nprev, f), jnp.float32),
    )(M, p, base)


def _kron_lift(w):
    return jnp.kron(jnp.eye(128 // w.shape[0], dtype=w.dtype), w.T)


def kernel(x, adj, W1, b1, W2, b2, pW1, pb1, pW2, pb2, pW3, pb3, pW4, pb4,
           pW5, pb5):
    b, c, n, l = x.shape
    f = b * c * l

    w1b = [W1[:, i * c:(i + 1) * c] for i in range(GDEP + 1)]
    w2b = [W2[:, i * c:(i + 1) * c] for i in range(GDEP + 1)]
    k1cat = jnp.concatenate([_kron_lift(w) for w in w1b], axis=0)
    k2cat = jnp.concatenate([_kron_lift(w) for w in w2b], axis=0)
    bias1 = jnp.tile(b1, b * l)[None, :]
    bias2 = jnp.tile(b2, b * l)[None, :]
    pws, pbs = [], []
    for pW, pb in ((pW1, pb1), (pW2, pb2), (pW3, pb3), (pW4, pb4), (pW5, pb5)):
        pws.append(pW.reshape(b, c, l).transpose(0, 2, 1).reshape(f, 1))
        pbs.append(pb.reshape(1, 1))

    H = x.transpose(2, 0, 3, 1).reshape(n, f)

    g = adj
    n_real = n
    Xcur = H
    hs, Ms = [], []
    for lvl in range(5):
        npad = _pad_to(n_real)
        if g.shape[0] != npad:
            g = jnp.pad(g, ((0, npad - g.shape[0]), (0, npad - g.shape[1])))
        if Xcur.shape[0] != npad:
            Xcur = jnp.pad(Xcur, ((0, npad - Xcur.shape[0]), (0, 0)))

        gT = _transpose(g)
        eye = jnp.eye(npad, dtype=g.dtype)
        dr = jnp.sum(g + eye, axis=1).reshape(npad, 1)
        dc = jnp.sum(g.T + eye, axis=1).reshape(npad, 1)
        a, at = _normadj(g, gT, dr, dc)

        h1 = _propagate(a, Xcur, Xcur)
        h2 = _propagate(a, h1, Xcur)
        k1 = _propagate(at, Xcur, Xcur)
        k2 = _propagate(at, k1, Xcur)

        C, s = _mix_scores([Xcur, h1, h2, k1, k2], [k1cat, k2cat],
                           [bias1, bias2], pws[lvl], pbs[lvl])

        kk_real = max(2, int(KS[lvl] * n_real))
        kkp = _pad_to(kk_real)
        srow = s.reshape(1, npad)
        rank = _rank(s, srow, n_real)
        idx, vals, M = _select(rank.reshape(1, npad), srow, kk_real, kkp)

        idx_flat = idx.reshape(kkp)
        Craw, gsel, gtsel = _sc_gather(idx_flat, [C, g, gT])
        Csel = _scale(Craw, vals)

        un, deg = _binmm(gsel, gtsel, kk_real)
        g = _rownorm(un, deg)

        hs.append(Csel)
        Ms.append(M)
        Xcur = Csel
        n_real = kk_real

    P = hs[4]
    for lvl in (3, 2, 1, 0):
        P = _unpool_add(hs[lvl], P, Ms[lvl + 1])
    out_flat = _unpool_add(jnp.zeros((n, f), jnp.float32), P, Ms[0])

    return out_flat.reshape(n, b, l, c).transpose(1, 3, 0, 2)


# speedup vs baseline: 1.1821x; 1.1821x over previous
import functools
from functools import partial

import jax
import jax.numpy as jnp
from jax import lax
from jax.experimental import pallas as pl
from jax.experimental.pallas import tpu as pltpu
from jax.experimental.pallas import tpu_sc as plsc

GDEP = 2
ALPHA = 0.05
KS = [0.9, 0.8, 0.7, 0.6, 0.5]

_HI = jax.lax.Precision.HIGHEST
BT = 128


PADU = 256


def _pad_to(v):
    return ((v + PADU - 1) // PADU) * PADU


def _transpose_body(i_ref, o_ref):
    o_ref[...] = jnp.swapaxes(i_ref[...], 0, 1)


def _transpose(g):
    n = g.shape[0]
    gr = n // BT
    return pl.pallas_call(
        _transpose_body,
        grid=(gr, gr),
        in_specs=[pl.BlockSpec((BT, BT), lambda i, j: (i, j))],
        out_specs=pl.BlockSpec((BT, BT), lambda i, j: (j, i)),
        out_shape=jax.ShapeDtypeStruct((n, n), g.dtype),
    )(g)


def _normadj_body(g_ref, gt_ref, dr_ref, dc_ref, a_ref, at_ref):
    i = pl.program_id(0)
    j = pl.program_id(1)
    rows = i * BT + jax.lax.broadcasted_iota(jnp.int32, (BT, BT), 0)
    cols = j * BT + jax.lax.broadcasted_iota(jnp.int32, (BT, BT), 1)
    eye = (rows == cols).astype(jnp.float32)
    a_ref[...] = (g_ref[...] + eye) / dr_ref[...]
    at_ref[...] = (gt_ref[...] + eye) / dc_ref[...]


def _normadj(g, gt, dr, dc):
    n = g.shape[0]
    spec = pl.BlockSpec((BT, BT), lambda i, j: (i, j))
    dspec = pl.BlockSpec((BT, 1), lambda i, j: (i, 0))
    return pl.pallas_call(
        _normadj_body,
        grid=(n // BT, n // BT),
        in_specs=[spec, spec, dspec, dspec],
        out_specs=[spec, spec],
        out_shape=[jax.ShapeDtypeStruct((n, n), jnp.float32),
                   jax.ShapeDtypeStruct((n, n), jnp.float32)],
    )(g, gt, dr, dc)


BP = 256


def _prop2_body(a_ref, at_ref, xh_ref, xk_ref, x0_ref, oh_ref, ok_ref):
    x0 = x0_ref[...]
    acc_h = jnp.dot(a_ref[...].astype(jnp.bfloat16),
                    xh_ref[...].astype(jnp.bfloat16),
                    preferred_element_type=jnp.float32)
    oh_ref[...] = ALPHA * x0 + (1.0 - ALPHA) * acc_h
    acc_k = jnp.dot(at_ref[...].astype(jnp.bfloat16),
                    xk_ref[...].astype(jnp.bfloat16),
                    preferred_element_type=jnp.float32)
    ok_ref[...] = ALPHA * x0 + (1.0 - ALPHA) * acc_k


def _propagate2(a, at, xh, xk, x0):
    n = a.shape[0]
    f = xh.shape[1]
    outs = pl.pallas_call(
        _prop2_body,
        grid=(n // BP, f // BP),
        in_specs=[
            pl.BlockSpec((BP, n), lambda m, c: (m, 0)),
            pl.BlockSpec((BP, n), lambda m, c: (m, 0)),
            pl.BlockSpec((n, BP), lambda m, c: (0, c)),
            pl.BlockSpec((n, BP), lambda m, c: (0, c)),
            pl.BlockSpec((BP, BP), lambda m, c: (m, c)),
        ],
        out_specs=[pl.BlockSpec((BP, BP), lambda m, c: (m, c)),
                   pl.BlockSpec((BP, BP), lambda m, c: (m, c))],
        out_shape=[jax.ShapeDtypeStruct((n, f), jnp.float32),
                   jax.ShapeDtypeStruct((n, f), jnp.float32)],
    )(a, at, xh, xk, x0)
    return outs[0], outs[1]


def _mix_body(x0, x1, x2, x3, x4, k1_ref, k2_ref, b1_ref, b2_ref, pw_ref,
              pb_ref, c_ref, s_ref, *, nc):
    c = pl.program_id(1)
    cat1 = jnp.concatenate([x0[...], x1[...], x2[...]], axis=1)
    cat2 = jnp.concatenate([x0[...], x3[...], x4[...]], axis=1)
    m1 = jnp.dot(cat1.astype(jnp.bfloat16), k1_ref[...].astype(jnp.bfloat16),
                 preferred_element_type=jnp.float32)
    m2 = jnp.dot(cat2.astype(jnp.bfloat16), k2_ref[...].astype(jnp.bfloat16),
                 preferred_element_type=jnp.float32)
    cb = jnp.maximum((m1 + b1_ref[...]) + (m2 + b2_ref[...]), 0.0)
    c_ref[...] = cb
    part = jnp.dot(cb.astype(jnp.bfloat16), pw_ref[...].astype(jnp.bfloat16),
                   preferred_element_type=jnp.float32)
    tot = jnp.where(c == 0, part, s_ref[...] + part)
    s_ref[...] = jnp.where(c == nc - 1,
                           jax.nn.sigmoid(tot + pb_ref[...]), tot)


def _mix_scores(xs, ks, biases, pw, pb):
    n, f = xs[0].shape
    nc = f // BT
    xspec = pl.BlockSpec((BT, BT), lambda m, c: (m, c))
    kspec = pl.BlockSpec((3 * BT, BT), lambda m, c: (0, 0))
    bspec = pl.BlockSpec((1, BT), lambda m, c: (0, c))
    return pl.pallas_call(
        partial(_mix_body, nc=nc),
        grid=(n // BT, nc),
        in_specs=[xspec] * 5 + [kspec] * 2 + [bspec] * 2 + [
            pl.BlockSpec((BT, 1), lambda m, c: (c, 0)),
            pl.BlockSpec((1, 1), lambda m, c: (0, 0)),
        ],
        out_specs=[
            pl.BlockSpec((BT, BT), lambda m, c: (m, c)),
            pl.BlockSpec((BT, 1), lambda m, c: (m, 0)),
        ],
        out_shape=[
            jax.ShapeDtypeStruct((n, f), jnp.float32),
            jax.ShapeDtypeStruct((n, 1), jnp.float32),
        ],
    )(*xs, *ks, *biases, pw, pb)


def _rank_body(s_ref, sall_ref, r_ref, *, n_real, npad):
    i0 = pl.program_id(0) * BT
    si = s_ref[...]
    sj = sall_ref[...]
    rows = i0 + jax.lax.broadcasted_iota(jnp.int32, (BT, npad), 0)
    cols = jax.lax.broadcasted_iota(jnp.int32, (BT, npad), 1)
    vj = cols < n_real
    gt = (sj > si) & vj
    eq = (sj == si) & (cols < rows) & vj
    rank = jnp.sum(gt.astype(jnp.int32) + eq.astype(jnp.int32),
                   axis=1, keepdims=True)
    rank = jnp.where(rows[:, :1] < n_real, rank, jnp.int32(2**30))
    r_ref[...] = rank


def _rank(s, srow, n_real):
    npad = s.shape[0]
    return pl.pallas_call(
        partial(_rank_body, n_real=n_real, npad=npad),
        grid=(npad // BT,),
        in_specs=[
            pl.BlockSpec((BT, 1), lambda i: (i, 0)),
            pl.BlockSpec((1, npad), lambda i: (0, 0)),
        ],
        out_specs=pl.BlockSpec((BT, 1), lambda i: (i, 0)),
        out_shape=jax.ShapeDtypeStruct((npad, 1), jnp.int32),
    )(s, srow)


def _select_body(rrow_ref, srow_ref, i_ref, v_ref, m_ref, *, kk_real, npad):
    r0 = pl.program_id(0) * BT
    rglob = r0 + jax.lax.broadcasted_iota(jnp.int32, (BT, npad), 0)
    cols = jax.lax.broadcasted_iota(jnp.int32, (BT, npad), 1)
    onehot = (rrow_ref[...] == rglob) & (rglob < kk_real)
    valid = rglob[:, :1] < kk_real
    idx = jnp.sum(jnp.where(onehot, cols, 0), axis=1, keepdims=True)
    val = jnp.sum(jnp.where(onehot, srow_ref[...], 0.0), axis=1, keepdims=True)
    i_ref[...] = jnp.where(valid, idx, 0)
    v_ref[...] = jnp.where(valid, val, 0.0)
    m_ref[...] = onehot.astype(jnp.float32)


def _select(rrow, srow, kk_real, kkp):
    npad = rrow.shape[1]
    return pl.pallas_call(
        partial(_select_body, kk_real=kk_real, npad=npad),
        grid=(kkp // BT,),
        in_specs=[
            pl.BlockSpec((1, npad), lambda i: (0, 0)),
            pl.BlockSpec((1, npad), lambda i: (0, 0)),
        ],
        out_specs=[
            pl.BlockSpec((BT, 1), lambda i: (i, 0)),
            pl.BlockSpec((BT, 1), lambda i: (i, 0)),
            pl.BlockSpec((BT, npad), lambda i: (i, 0)),
        ],
        out_shape=[
            jax.ShapeDtypeStruct((kkp, 1), jnp.int32),
            jax.ShapeDtypeStruct((kkp, 1), jnp.float32),
            jax.ShapeDtypeStruct((kkp, npad), jnp.float32),
        ],
    )(rrow, srow)


def _sc_gather(idx, tables):
    kkp = idx.shape[0]
    NC, NS = 2, 16
    bpw = kkp // (NC * NS)
    CH = 8
    nt = len(tables)
    mesh = plsc.VectorSubcoreMesh(core_axis_name="c", subcore_axis_name="s")
    outs = [jax.ShapeDtypeStruct((kkp, t.shape[1]), t.dtype) for t in tables]
    scr = ([pltpu.VMEM((CH,), jnp.int32)]
           + [pltpu.VMEM((CH, t.shape[1]), t.dtype) for t in tables]
           + [pltpu.SemaphoreType.DMA])

    @functools.partial(pl.kernel, mesh=mesh, out_type=outs, scratch_types=scr)
    def k(*refs):
        idx_hbm = refs[0]
        tab = refs[1:1 + nt]
        out = refs[1 + nt:1 + 2 * nt]
        idx_v = refs[1 + 2 * nt]
        bufs = refs[2 + 2 * nt:2 + 3 * nt]
        sem = refs[2 + 3 * nt]
        wid = lax.axis_index("s") * NC + lax.axis_index("c")
        base = wid * bpw
        for ch in range(bpw // CH):
            off = base + ch * CH
            pltpu.sync_copy(idx_hbm.at[pl.ds(off, CH)], idx_v)
            for t in range(nt):
                pltpu.async_copy(tab[t].at[idx_v], bufs[t], sem).wait()
                pltpu.sync_copy(bufs[t], out[t].at[pl.ds(off, CH)])

    res = k(idx, *tables)
    return tuple(res) if isinstance(res, (list, tuple)) else (res,)


def _scale_body(c_ref, v_ref, o_ref):
    o_ref[...] = c_ref[...] * v_ref[...]


def _scale(csel, vals):
    kkp, f = csel.shape
    return pl.pallas_call(
        _scale_body,
        grid=(kkp // BT,),
        in_specs=[pl.BlockSpec((BT, f), lambda i: (i, 0)),
                  pl.BlockSpec((BT, 1), lambda i: (i, 0))],
        out_specs=pl.BlockSpec((BT, f), lambda i: (i, 0)),
        out_shape=jax.ShapeDtypeStruct((kkp, f), jnp.float32),
    )(csel, vals)


def _binmm_body(a_ref, b_ref, un_ref, deg_ref, acc_ref, *, nk, kk_real):
    k = pl.program_id(2)
    j = pl.program_id(1)
    m = pl.program_id(0)

    @pl.when(k == 0)
    def _():
        acc_ref[...] = jnp.zeros_like(acc_ref)

    arows = m * BT + jax.lax.broadcasted_iota(jnp.int32, (BT, BT), 0)
    brows = j * BT + jax.lax.broadcasted_iota(jnp.int32, (BT, BT), 0)
    a = ((a_ref[...] != 0) & (arows < kk_real)).astype(jnp.bfloat16)
    b = ((b_ref[...] != 0) & (brows < kk_real)).astype(jnp.bfloat16)
    acc_ref[...] += jax.lax.dot_general(
        a, b, (((1,), (1,)), ((), ())),
        preferred_element_type=jnp.float32)

    @pl.when(k == nk - 1)
    def _():
        un = (acc_ref[...] != 0).astype(jnp.float32)
        un_ref[...] = un
        part = jnp.sum(un, axis=1, keepdims=True)
        deg_ref[...] = jnp.where(j == 0, part, deg_ref[...] + part)


def _binmm(gsel, gtsel, kk_real):
    kkp, npad = gsel.shape
    nk = npad // BT
    return pl.pallas_call(
        partial(_binmm_body, nk=nk, kk_real=kk_real),
        grid=(kkp // BT, kkp // BT, nk),
        in_specs=[
            pl.BlockSpec((BT, BT), lambda m, j, k: (m, k)),
            pl.BlockSpec((BT, BT), lambda m, j, k: (j, k)),
        ],
        out_specs=[
            pl.BlockSpec((BT, BT), lambda m, j, k: (m, j)),
            pl.BlockSpec((BT, 1), lambda m, j, k: (m, 0)),
        ],
        out_shape=[
            jax.ShapeDtypeStruct((kkp, kkp), jnp.float32),
            jax.ShapeDtypeStruct((kkp, 1), jnp.float32),
        ],
        scratch_shapes=[pltpu.VMEM((BT, BT), jnp.float32)],
    )(gsel, gtsel)


def _rownorm_body(un_ref, deg_ref, o_ref):
    d = deg_ref[...]
    o_ref[...] = jnp.where(d > 0, un_ref[...] / d, 0.0)


def _rownorm(un, deg):
    kkp = un.shape[0]
    return pl.pallas_call(
        _rownorm_body,
        grid=(kkp // BT,),
        in_specs=[
            pl.BlockSpec((BT, kkp), lambda i: (i, 0)),
            pl.BlockSpec((BT, 1), lambda i: (i, 0)),
        ],
        out_specs=pl.BlockSpec((BT, kkp), lambda i: (i, 0)),
        out_shape=jax.ShapeDtypeStruct((kkp, kkp), jnp.float32),
    )(un, deg)


def _unpool_body(m_ref, p_ref, base_ref, o_ref):
    acc = jax.lax.dot_general(
        m_ref[...].astype(jnp.bfloat16), p_ref[...].astype(jnp.bfloat16),
        (((0,), (0,)), ((), ())), preferred_element_type=jnp.float32)
    o_ref[...] = base_ref[...] + acc


def _unpool_add(base, p, M):
    nprev, f = base.shape
    kkp = M.shape[0]
    return pl.pallas_call(
        _unpool_body,
        grid=(nprev // BT, f // BT),
        in_specs=[
            pl.BlockSpec((kkp, BT), lambda m, c: (0, m)),
            pl.BlockSpec((kkp, BT), lambda m, c: (0, c)),
            pl.BlockSpec((BT, BT), lambda m, c: (m, c)),
        ],
        out_specs=pl.BlockSpec((BT, BT), lambda m, c: (m, c)),
        out_shape=jax.ShapeDtypeStruct((nprev, f), jnp.float32),
    )(M, p, base)


def _kron_lift(w):
    return jnp.kron(jnp.eye(128 // w.shape[0], dtype=w.dtype), w.T)


def kernel(x, adj, W1, b1, W2, b2, pW1, pb1, pW2, pb2, pW3, pb3, pW4, pb4,
           pW5, pb5):
    b, c, n, l = x.shape
    f = b * c * l

    w1b = [W1[:, i * c:(i + 1) * c] for i in range(GDEP + 1)]
    w2b = [W2[:, i * c:(i + 1) * c] for i in range(GDEP + 1)]
    k1cat = jnp.concatenate([_kron_lift(w) for w in w1b], axis=0)
    k2cat = jnp.concatenate([_kron_lift(w) for w in w2b], axis=0)
    bias1 = jnp.tile(b1, b * l)[None, :]
    bias2 = jnp.tile(b2, b * l)[None, :]
    pws, pbs = [], []
    for pW, pb in ((pW1, pb1), (pW2, pb2), (pW3, pb3), (pW4, pb4), (pW5, pb5)):
        pws.append(pW.reshape(b, c, l).transpose(0, 2, 1).reshape(f, 1))
        pbs.append(pb.reshape(1, 1))

    H = x.transpose(2, 0, 3, 1).reshape(n, f)

    g = adj
    n_real = n
    Xcur = H
    hs, Ms = [], []
    for lvl in range(5):
        npad = _pad_to(n_real)
        if g.shape[0] != npad:
            g = jnp.pad(g, ((0, npad - g.shape[0]), (0, npad - g.shape[1])))
        if Xcur.shape[0] != npad:
            Xcur = jnp.pad(Xcur, ((0, npad - Xcur.shape[0]), (0, 0)))

        gT = _transpose(g)
        eye = jnp.eye(npad, dtype=g.dtype)
        dr = jnp.sum(g + eye, axis=1).reshape(npad, 1)
        dc = jnp.sum(g.T + eye, axis=1).reshape(npad, 1)
        a, at = _normadj(g, gT, dr, dc)

        h1, k1 = _propagate2(a, at, Xcur, Xcur, Xcur)
        h2, k2 = _propagate2(a, at, h1, k1, Xcur)

        C, s = _mix_scores([Xcur, h1, h2, k1, k2], [k1cat, k2cat],
                           [bias1, bias2], pws[lvl], pbs[lvl])

        kk_real = max(2, int(KS[lvl] * n_real))
        kkp = _pad_to(kk_real)
        srow = s.reshape(1, npad)
        rank = _rank(s, srow, n_real)
        idx, vals, M = _select(rank.reshape(1, npad), srow, kk_real, kkp)

        idx_flat = idx.reshape(kkp)
        Craw, gsel, gtsel = _sc_gather(idx_flat, [C, g, gT])
        Csel = _scale(Craw, vals)

        un, deg = _binmm(gsel, gtsel, kk_real)
        g = _rownorm(un, deg)

        hs.append(Csel)
        Ms.append(M)
        Xcur = Csel
        n_real = kk_real

    P = hs[4]
    for lvl in (3, 2, 1, 0):
        P = _unpool_add(hs[lvl], P, Ms[lvl + 1])
    out_flat = _unpool_add(jnp.zeros((n, f), jnp.float32), P, Ms[0])

    return out_flat.reshape(n, b, l, c).transpose(1, 3, 0, 2)


# speedup vs baseline: 3.4637x; 2.9300x over previous
import functools
from functools import partial

import jax
import jax.numpy as jnp
from jax import lax
from jax.experimental import pallas as pl
from jax.experimental.pallas import tpu as pltpu
from jax.experimental.pallas import tpu_sc as plsc

GDEP = 2
ALPHA = 0.05
KS = [0.9, 0.8, 0.7, 0.6, 0.5]

_HI = jax.lax.Precision.HIGHEST
BT = 128
BP = 256


PADU = 256


def _pad_to(v):
    return ((v + PADU - 1) // PADU) * PADU


def _transpose_body(i_ref, o_ref):
    o_ref[...] = jnp.swapaxes(i_ref[...], 0, 1)


def _transpose(g):
    n = g.shape[0]
    gr = n // BP
    return pl.pallas_call(
        _transpose_body,
        grid=(gr, gr),
        in_specs=[pl.BlockSpec((BP, BP), lambda i, j: (i, j))],
        out_specs=pl.BlockSpec((BP, BP), lambda i, j: (j, i)),
        out_shape=jax.ShapeDtypeStruct((n, n), g.dtype),
    )(g)


def _normadj_body(g_ref, gt_ref, dr_ref, dc_ref, a_ref, at_ref):
    i = pl.program_id(0)
    j = pl.program_id(1)
    rows = i * BP + jax.lax.broadcasted_iota(jnp.int32, (BP, BP), 0)
    cols = j * BP + jax.lax.broadcasted_iota(jnp.int32, (BP, BP), 1)
    eye = (rows == cols).astype(jnp.float32)
    a_ref[...] = (g_ref[...] + eye) / dr_ref[...]
    at_ref[...] = (gt_ref[...] + eye) / dc_ref[...]


def _normadj(g, gt, dr, dc):
    n = g.shape[0]
    spec = pl.BlockSpec((BP, BP), lambda i, j: (i, j))
    dspec = pl.BlockSpec((BP, 1), lambda i, j: (i, 0))
    return pl.pallas_call(
        _normadj_body,
        grid=(n // BP, n // BP),
        in_specs=[spec, spec, dspec, dspec],
        out_specs=[spec, spec],
        out_shape=[jax.ShapeDtypeStruct((n, n), jnp.float32),
                   jax.ShapeDtypeStruct((n, n), jnp.float32)],
    )(g, gt, dr, dc)


def _prop2_body(a_ref, at_ref, xh_ref, xk_ref, x0_ref, oh_ref, ok_ref):
    x0 = x0_ref[...]
    acc_h = jnp.dot(a_ref[...].astype(jnp.bfloat16),
                    xh_ref[...].astype(jnp.bfloat16),
                    preferred_element_type=jnp.float32)
    oh_ref[...] = ALPHA * x0 + (1.0 - ALPHA) * acc_h
    acc_k = jnp.dot(at_ref[...].astype(jnp.bfloat16),
                    xk_ref[...].astype(jnp.bfloat16),
                    preferred_element_type=jnp.float32)
    ok_ref[...] = ALPHA * x0 + (1.0 - ALPHA) * acc_k


def _propagate2(a, at, xh, xk, x0):
    n = a.shape[0]
    f = xh.shape[1]
    outs = pl.pallas_call(
        _prop2_body,
        grid=(n // BP, f // BP),
        in_specs=[
            pl.BlockSpec((BP, n), lambda m, c: (m, 0)),
            pl.BlockSpec((BP, n), lambda m, c: (m, 0)),
            pl.BlockSpec((n, BP), lambda m, c: (0, c)),
            pl.BlockSpec((n, BP), lambda m, c: (0, c)),
            pl.BlockSpec((BP, BP), lambda m, c: (m, c)),
        ],
        out_specs=[pl.BlockSpec((BP, BP), lambda m, c: (m, c)),
                   pl.BlockSpec((BP, BP), lambda m, c: (m, c))],
        out_shape=[jax.ShapeDtypeStruct((n, f), jnp.float32),
                   jax.ShapeDtypeStruct((n, f), jnp.float32)],
    )(a, at, xh, xk, x0)
    return outs[0], outs[1]


def _mix_body(x0, x1, x2, x3, x4, k1_ref, k2_ref, b1_ref, b2_ref, pw_ref,
              pb_ref, c_ref, s_ref, *, nc):
    c = pl.program_id(1)
    cat1 = jnp.concatenate([x0[...], x1[...], x2[...]], axis=1)
    cat2 = jnp.concatenate([x0[...], x3[...], x4[...]], axis=1)
    m1 = jnp.dot(cat1.astype(jnp.bfloat16), k1_ref[...].astype(jnp.bfloat16),
                 preferred_element_type=jnp.float32)
    m2 = jnp.dot(cat2.astype(jnp.bfloat16), k2_ref[...].astype(jnp.bfloat16),
                 preferred_element_type=jnp.float32)
    cb = jnp.maximum((m1 + b1_ref[...]) + (m2 + b2_ref[...]), 0.0)
    c_ref[...] = cb
    part = jnp.dot(cb.astype(jnp.bfloat16), pw_ref[...].astype(jnp.bfloat16),
                   preferred_element_type=jnp.float32)
    tot = jnp.where(c == 0, part, s_ref[...] + part)
    s_ref[...] = jnp.where(c == nc - 1,
                           jax.nn.sigmoid(tot + pb_ref[...]), tot)


def _mix_scores(xs, ks, biases, pw, pb):
    n, f = xs[0].shape
    nc = f // BT
    xspec = pl.BlockSpec((BP, BT), lambda m, c: (m, c))
    kspec = pl.BlockSpec((3 * BT, BT), lambda m, c: (0, 0))
    bspec = pl.BlockSpec((1, BT), lambda m, c: (0, c))
    return pl.pallas_call(
        partial(_mix_body, nc=nc),
        grid=(n // BP, nc),
        in_specs=[xspec] * 5 + [kspec] * 2 + [bspec] * 2 + [
            pl.BlockSpec((BT, 1), lambda m, c: (c, 0)),
            pl.BlockSpec((1, 1), lambda m, c: (0, 0)),
        ],
        out_specs=[
            pl.BlockSpec((BP, BT), lambda m, c: (m, c)),
            pl.BlockSpec((BP, 1), lambda m, c: (m, 0)),
        ],
        out_shape=[
            jax.ShapeDtypeStruct((n, f), jnp.float32),
            jax.ShapeDtypeStruct((n, 1), jnp.float32),
        ],
    )(*xs, *ks, *biases, pw, pb)


def _rank_body(s_ref, sall_ref, r_ref, *, n_real, npad):
    i0 = pl.program_id(0) * BT
    si = s_ref[...]
    sj = sall_ref[...]
    rows = i0 + jax.lax.broadcasted_iota(jnp.int32, (BT, npad), 0)
    cols = jax.lax.broadcasted_iota(jnp.int32, (BT, npad), 1)
    vj = cols < n_real
    gt = (sj > si) & vj
    eq = (sj == si) & (cols < rows) & vj
    rank = jnp.sum(gt.astype(jnp.int32) + eq.astype(jnp.int32),
                   axis=1, keepdims=True)
    rank = jnp.where(rows[:, :1] < n_real, rank, jnp.int32(2**30))
    r_ref[...] = rank


def _rank(s, srow, n_real):
    npad = s.shape[0]
    return pl.pallas_call(
        partial(_rank_body, n_real=n_real, npad=npad),
        grid=(npad // BT,),
        in_specs=[
            pl.BlockSpec((BT, 1), lambda i: (i, 0)),
            pl.BlockSpec((1, npad), lambda i: (0, 0)),
        ],
        out_specs=pl.BlockSpec((BT, 1), lambda i: (i, 0)),
        out_shape=jax.ShapeDtypeStruct((npad, 1), jnp.int32),
    )(s, srow)


def _select_body(rrow_ref, srow_ref, i_ref, v_ref, m_ref, *, kk_real, npad):
    r0 = pl.program_id(0) * BT
    rglob = r0 + jax.lax.broadcasted_iota(jnp.int32, (BT, npad), 0)
    cols = jax.lax.broadcasted_iota(jnp.int32, (BT, npad), 1)
    onehot = (rrow_ref[...] == rglob) & (rglob < kk_real)
    valid = rglob[:, :1] < kk_real
    idx = jnp.sum(jnp.where(onehot, cols, 0), axis=1, keepdims=True)
    val = jnp.sum(jnp.where(onehot, srow_ref[...], 0.0), axis=1, keepdims=True)
    i_ref[...] = jnp.where(valid, idx, 0)
    v_ref[...] = jnp.where(valid, val, 0.0)
    m_ref[...] = onehot.astype(jnp.float32)


def _select(rrow, srow, kk_real, kkp):
    npad = rrow.shape[1]
    return pl.pallas_call(
        partial(_select_body, kk_real=kk_real, npad=npad),
        grid=(kkp // BT,),
        in_specs=[
            pl.BlockSpec((1, npad), lambda i: (0, 0)),
            pl.BlockSpec((1, npad), lambda i: (0, 0)),
        ],
        out_specs=[
            pl.BlockSpec((BT, 1), lambda i: (i, 0)),
            pl.BlockSpec((BT, 1), lambda i: (i, 0)),
            pl.BlockSpec((BT, npad), lambda i: (i, 0)),
        ],
        out_shape=[
            jax.ShapeDtypeStruct((kkp, 1), jnp.int32),
            jax.ShapeDtypeStruct((kkp, 1), jnp.float32),
            jax.ShapeDtypeStruct((kkp, npad), jnp.float32),
        ],
    )(rrow, srow)


def _sc_gather(idx, tables):
    kkp = idx.shape[0]
    NC, NS = 2, 16
    bpw = kkp // (NC * NS)
    CH = 8
    nt = len(tables)
    mesh = plsc.VectorSubcoreMesh(core_axis_name="c", subcore_axis_name="s")
    outs = [jax.ShapeDtypeStruct((kkp, t.shape[1]), t.dtype) for t in tables]
    scr = ([pltpu.VMEM((CH,), jnp.int32)]
           + [pltpu.VMEM((CH, t.shape[1]), t.dtype) for t in tables]
           + [pltpu.SemaphoreType.DMA])

    @functools.partial(pl.kernel, mesh=mesh, out_type=outs, scratch_types=scr)
    def k(*refs):
        idx_hbm = refs[0]
        tab = refs[1:1 + nt]
        out = refs[1 + nt:1 + 2 * nt]
        idx_v = refs[1 + 2 * nt]
        bufs = refs[2 + 2 * nt:2 + 3 * nt]
        sem = refs[2 + 3 * nt]
        wid = lax.axis_index("s") * NC + lax.axis_index("c")
        base = wid * bpw
        for ch in range(bpw // CH):
            off = base + ch * CH
            pltpu.sync_copy(idx_hbm.at[pl.ds(off, CH)], idx_v)
            for t in range(nt):
                pltpu.async_copy(tab[t].at[idx_v], bufs[t], sem).wait()
                pltpu.sync_copy(bufs[t], out[t].at[pl.ds(off, CH)])

    res = k(idx, *tables)
    return tuple(res) if isinstance(res, (list, tuple)) else (res,)


def _scale_body(c_ref, v_ref, o_ref):
    o_ref[...] = c_ref[...] * v_ref[...]


def _scale(csel, vals):
    kkp, f = csel.shape
    return pl.pallas_call(
        _scale_body,
        grid=(kkp // BT,),
        in_specs=[pl.BlockSpec((BT, f), lambda i: (i, 0)),
                  pl.BlockSpec((BT, 1), lambda i: (i, 0))],
        out_specs=pl.BlockSpec((BT, f), lambda i: (i, 0)),
        out_shape=jax.ShapeDtypeStruct((kkp, f), jnp.float32),
    )(csel, vals)


def _binmm_body(a_ref, b_ref, un_ref, deg_ref, acc_ref, *, nk, kk_real):
    k = pl.program_id(2)
    j = pl.program_id(1)
    m = pl.program_id(0)

    @pl.when(k == 0)
    def _():
        acc_ref[...] = jnp.zeros_like(acc_ref)

    arows = m * BP + jax.lax.broadcasted_iota(jnp.int32, (BP, BP), 0)
    brows = j * BP + jax.lax.broadcasted_iota(jnp.int32, (BP, BP), 0)
    a = ((a_ref[...] != 0) & (arows < kk_real)).astype(jnp.bfloat16)
    b = ((b_ref[...] != 0) & (brows < kk_real)).astype(jnp.bfloat16)
    acc_ref[...] += jax.lax.dot_general(
        a, b, (((1,), (1,)), ((), ())),
        preferred_element_type=jnp.float32)

    @pl.when(k == nk - 1)
    def _():
        un = (acc_ref[...] != 0).astype(jnp.float32)
        un_ref[...] = un
        part = jnp.sum(un, axis=1, keepdims=True)
        deg_ref[...] = jnp.where(j == 0, part, deg_ref[...] + part)


def _binmm(gsel, gtsel, kk_real):
    kkp, npad = gsel.shape
    nk = npad // BP
    return pl.pallas_call(
        partial(_binmm_body, nk=nk, kk_real=kk_real),
        grid=(kkp // BP, kkp // BP, nk),
        in_specs=[
            pl.BlockSpec((BP, BP), lambda m, j, k: (m, k)),
            pl.BlockSpec((BP, BP), lambda m, j, k: (j, k)),
        ],
        out_specs=[
            pl.BlockSpec((BP, BP), lambda m, j, k: (m, j)),
            pl.BlockSpec((BP, 1), lambda m, j, k: (m, 0)),
        ],
        out_shape=[
            jax.ShapeDtypeStruct((kkp, kkp), jnp.float32),
            jax.ShapeDtypeStruct((kkp, 1), jnp.float32),
        ],
        scratch_shapes=[pltpu.VMEM((BP, BP), jnp.float32)],
    )(gsel, gtsel)


def _rownorm_body(un_ref, deg_ref, o_ref):
    d = deg_ref[...]
    o_ref[...] = jnp.where(d > 0, un_ref[...] / d, 0.0)


def _rownorm(un, deg):
    kkp = un.shape[0]
    return pl.pallas_call(
        _rownorm_body,
        grid=(kkp // BT,),
        in_specs=[
            pl.BlockSpec((BT, kkp), lambda i: (i, 0)),
            pl.BlockSpec((BT, 1), lambda i: (i, 0)),
        ],
        out_specs=pl.BlockSpec((BT, kkp), lambda i: (i, 0)),
        out_shape=jax.ShapeDtypeStruct((kkp, kkp), jnp.float32),
    )(un, deg)


def _unpool_body(m_ref, p_ref, base_ref, o_ref):
    acc = jax.lax.dot_general(
        m_ref[...].astype(jnp.bfloat16), p_ref[...].astype(jnp.bfloat16),
        (((0,), (0,)), ((), ())), preferred_element_type=jnp.float32)
    o_ref[...] = base_ref[...] + acc


def _unpool_add(base, p, M):
    nprev, f = base.shape
    kkp = M.shape[0]
    return pl.pallas_call(
        _unpool_body,
        grid=(nprev // BP, f // BP),
        in_specs=[
            pl.BlockSpec((kkp, BP), lambda m, c: (0, m)),
            pl.BlockSpec((kkp, BP), lambda m, c: (0, c)),
            pl.BlockSpec((BP, BP), lambda m, c: (m, c)),
        ],
        out_specs=pl.BlockSpec((BP, BP), lambda m, c: (m, c)),
        out_shape=jax.ShapeDtypeStruct((nprev, f), jnp.float32),
    )(M, p, base)


def _kron_lift(w):
    return jnp.kron(jnp.eye(128 // w.shape[0], dtype=w.dtype), w.T)


def kernel(x, adj, W1, b1, W2, b2, pW1, pb1, pW2, pb2, pW3, pb3, pW4, pb4,
           pW5, pb5):
    b, c, n, l = x.shape
    f = b * c * l

    w1b = [W1[:, i * c:(i + 1) * c] for i in range(GDEP + 1)]
    w2b = [W2[:, i * c:(i + 1) * c] for i in range(GDEP + 1)]
    k1cat = jnp.concatenate([_kron_lift(w) for w in w1b], axis=0)
    k2cat = jnp.concatenate([_kron_lift(w) for w in w2b], axis=0)
    bias1 = jnp.tile(b1, b * l)[None, :]
    bias2 = jnp.tile(b2, b * l)[None, :]
    pws, pbs = [], []
    for pW, pb in ((pW1, pb1), (pW2, pb2), (pW3, pb3), (pW4, pb4), (pW5, pb5)):
        pws.append(pW.reshape(b, c, l).transpose(0, 2, 1).reshape(f, 1))
        pbs.append(pb.reshape(1, 1))

    H = x.transpose(2, 0, 3, 1).reshape(n, f)

    g = adj
    n_real = n
    Xcur = H
    hs, Ms = [], []
    for lvl in range(5):
        npad = _pad_to(n_real)
        if g.shape[0] != npad:
            g = jnp.pad(g, ((0, npad - g.shape[0]), (0, npad - g.shape[1])))
        if Xcur.shape[0] != npad:
            Xcur = jnp.pad(Xcur, ((0, npad - Xcur.shape[0]), (0, 0)))

        gT = _transpose(g)
        eye = jnp.eye(npad, dtype=g.dtype)
        dr = jnp.sum(g + eye, axis=1).reshape(npad, 1)
        dc = jnp.sum(g.T + eye, axis=1).reshape(npad, 1)
        a, at = _normadj(g, gT, dr, dc)

        h1, k1 = _propagate2(a, at, Xcur, Xcur, Xcur)
        h2, k2 = _propagate2(a, at, h1, k1, Xcur)

        C, s = _mix_scores([Xcur, h1, h2, k1, k2], [k1cat, k2cat],
                           [bias1, bias2], pws[lvl], pbs[lvl])

        kk_real = max(2, int(KS[lvl] * n_real))
        kkp = _pad_to(kk_real)
        srow = s.reshape(1, npad)
        rank = _rank(s, srow, n_real)
        idx, vals, M = _select(rank.reshape(1, npad), srow, kk_real, kkp)

        idx_flat = idx.reshape(kkp)
        Craw, gsel, gtsel = _sc_gather(idx_flat, [C, g, gT])
        Csel = _scale(Craw, vals)

        un, deg = _binmm(gsel, gtsel, kk_real)
        g = _rownorm(un, deg)

        hs.append(Csel)
        Ms.append(M)
        Xcur = Csel
        n_real = kk_real

    P = hs[4]
    for lvl in (3, 2, 1, 0):
        P = _unpool_add(hs[lvl], P, Ms[lvl + 1])
    out_flat = _unpool_add(jnp.zeros((n, f), jnp.float32), P, Ms[0])

    return out_flat.reshape(n, b, l, c).transpose(1, 3, 0, 2)


# speedup vs baseline: 3.5219x; 1.0168x over previous
import functools
from functools import partial

import jax
import jax.numpy as jnp
from jax import lax
from jax.experimental import pallas as pl
from jax.experimental.pallas import tpu as pltpu
from jax.experimental.pallas import tpu_sc as plsc

GDEP = 2
ALPHA = 0.05
KS = [0.9, 0.8, 0.7, 0.6, 0.5]

_HI = jax.lax.Precision.HIGHEST
BT = 128
BP = 256


PADU = 256


def _pad_to(v):
    return ((v + PADU - 1) // PADU) * PADU


def _transpose_body(i_ref, o_ref):
    o_ref[...] = jnp.swapaxes(i_ref[...], 0, 1)


def _transpose(g):
    n = g.shape[0]
    gr = n // BP
    return pl.pallas_call(
        _transpose_body,
        grid=(gr, gr),
        in_specs=[pl.BlockSpec((BP, BP), lambda i, j: (i, j))],
        out_specs=pl.BlockSpec((BP, BP), lambda i, j: (j, i)),
        out_shape=jax.ShapeDtypeStruct((n, n), g.dtype),
    )(g)


def _normadj_body(g_ref, gt_ref, dr_ref, dc_ref, a_ref, at_ref):
    i = pl.program_id(0)
    j = pl.program_id(1)
    rows = i * BP + jax.lax.broadcasted_iota(jnp.int32, (BP, BP), 0)
    cols = j * BP + jax.lax.broadcasted_iota(jnp.int32, (BP, BP), 1)
    eye = (rows == cols).astype(jnp.float32)
    a_ref[...] = (g_ref[...] + eye) / dr_ref[...]
    at_ref[...] = (gt_ref[...] + eye) / dc_ref[...]


def _normadj(g, gt, dr, dc):
    n = g.shape[0]
    spec = pl.BlockSpec((BP, BP), lambda i, j: (i, j))
    dspec = pl.BlockSpec((BP, 1), lambda i, j: (i, 0))
    return pl.pallas_call(
        _normadj_body,
        grid=(n // BP, n // BP),
        in_specs=[spec, spec, dspec, dspec],
        out_specs=[spec, spec],
        out_shape=[jax.ShapeDtypeStruct((n, n), jnp.float32),
                   jax.ShapeDtypeStruct((n, n), jnp.float32)],
    )(g, gt, dr, dc)


def _prop2_body(a_ref, at_ref, xh_ref, xk_ref, x0_ref, oh_ref, ok_ref):
    x0 = x0_ref[...]
    acc_h = jnp.dot(a_ref[...].astype(jnp.bfloat16),
                    xh_ref[...].astype(jnp.bfloat16),
                    preferred_element_type=jnp.float32)
    oh_ref[...] = ALPHA * x0 + (1.0 - ALPHA) * acc_h
    acc_k = jnp.dot(at_ref[...].astype(jnp.bfloat16),
                    xk_ref[...].astype(jnp.bfloat16),
                    preferred_element_type=jnp.float32)
    ok_ref[...] = ALPHA * x0 + (1.0 - ALPHA) * acc_k


def _propagate2(a, at, xh, xk, x0):
    n = a.shape[0]
    f = xh.shape[1]
    outs = pl.pallas_call(
        _prop2_body,
        grid=(n // BP, f // BP),
        in_specs=[
            pl.BlockSpec((BP, n), lambda m, c: (m, 0)),
            pl.BlockSpec((BP, n), lambda m, c: (m, 0)),
            pl.BlockSpec((n, BP), lambda m, c: (0, c)),
            pl.BlockSpec((n, BP), lambda m, c: (0, c)),
            pl.BlockSpec((BP, BP), lambda m, c: (m, c)),
        ],
        out_specs=[pl.BlockSpec((BP, BP), lambda m, c: (m, c)),
                   pl.BlockSpec((BP, BP), lambda m, c: (m, c))],
        out_shape=[jax.ShapeDtypeStruct((n, f), jnp.float32),
                   jax.ShapeDtypeStruct((n, f), jnp.float32)],
    )(a, at, xh, xk, x0)
    return outs[0], outs[1]


def _mix_body(x0, x1, x2, x3, x4, k1_ref, k2_ref, b1_ref, b2_ref, pw_ref,
              pb_ref, c_ref, s_ref, *, nc):
    c = pl.program_id(1)
    cat1 = jnp.concatenate([x0[...], x1[...], x2[...]], axis=1)
    cat2 = jnp.concatenate([x0[...], x3[...], x4[...]], axis=1)
    m1 = jnp.dot(cat1.astype(jnp.bfloat16), k1_ref[...].astype(jnp.bfloat16),
                 preferred_element_type=jnp.float32)
    m2 = jnp.dot(cat2.astype(jnp.bfloat16), k2_ref[...].astype(jnp.bfloat16),
                 preferred_element_type=jnp.float32)
    cb = jnp.maximum((m1 + b1_ref[...]) + (m2 + b2_ref[...]), 0.0)
    c_ref[...] = cb
    part = jnp.dot(cb.astype(jnp.bfloat16), pw_ref[...].astype(jnp.bfloat16),
                   preferred_element_type=jnp.float32)
    tot = jnp.where(c == 0, part, s_ref[...] + part)
    s_ref[...] = jnp.where(c == nc - 1,
                           jax.nn.sigmoid(tot + pb_ref[...]), tot)


def _mix_scores(xs, ks, biases, pw, pb):
    n, f = xs[0].shape
    nc = f // BT
    xspec = pl.BlockSpec((BP, BT), lambda m, c: (m, c))
    kspec = pl.BlockSpec((3 * BT, BT), lambda m, c: (0, 0))
    bspec = pl.BlockSpec((1, BT), lambda m, c: (0, c))
    return pl.pallas_call(
        partial(_mix_body, nc=nc),
        grid=(n // BP, nc),
        in_specs=[xspec] * 5 + [kspec] * 2 + [bspec] * 2 + [
            pl.BlockSpec((BT, 1), lambda m, c: (c, 0)),
            pl.BlockSpec((1, 1), lambda m, c: (0, 0)),
        ],
        out_specs=[
            pl.BlockSpec((BP, BT), lambda m, c: (m, c)),
            pl.BlockSpec((BP, 1), lambda m, c: (m, 0)),
        ],
        out_shape=[
            jax.ShapeDtypeStruct((n, f), jnp.float32),
            jax.ShapeDtypeStruct((n, 1), jnp.float32),
        ],
    )(*xs, *ks, *biases, pw, pb)


def _rank_body(s_ref, sall_ref, r_ref, *, n_real, npad):
    i0 = pl.program_id(0) * BP
    si = s_ref[...]
    sj = sall_ref[...]
    rows = i0 + jax.lax.broadcasted_iota(jnp.int32, (BP, npad), 0)
    cols = jax.lax.broadcasted_iota(jnp.int32, (BP, npad), 1)
    vj = cols < n_real
    gt = (sj > si) & vj
    eq = (sj == si) & (cols < rows) & vj
    rank = jnp.sum(gt.astype(jnp.int32) + eq.astype(jnp.int32),
                   axis=1, keepdims=True)
    rank = jnp.where(rows[:, :1] < n_real, rank, jnp.int32(2**30))
    r_ref[...] = rank


def _rank(s, srow, n_real):
    npad = s.shape[0]
    return pl.pallas_call(
        partial(_rank_body, n_real=n_real, npad=npad),
        grid=(npad // BP,),
        in_specs=[
            pl.BlockSpec((BP, 1), lambda i: (i, 0)),
            pl.BlockSpec((1, npad), lambda i: (0, 0)),
        ],
        out_specs=pl.BlockSpec((BP, 1), lambda i: (i, 0)),
        out_shape=jax.ShapeDtypeStruct((npad, 1), jnp.int32),
    )(s, srow)


def _select_body(rrow_ref, srow_ref, i_ref, v_ref, m_ref, *, kk_real, npad):
    r0 = pl.program_id(0) * BP
    rglob = r0 + jax.lax.broadcasted_iota(jnp.int32, (BP, npad), 0)
    cols = jax.lax.broadcasted_iota(jnp.int32, (BP, npad), 1)
    onehot = (rrow_ref[...] == rglob) & (rglob < kk_real)
    valid = rglob[:, :1] < kk_real
    idx = jnp.sum(jnp.where(onehot, cols, 0), axis=1, keepdims=True)
    val = jnp.sum(jnp.where(onehot, srow_ref[...], 0.0), axis=1, keepdims=True)
    i_ref[...] = jnp.where(valid, idx, 0)
    v_ref[...] = jnp.where(valid, val, 0.0)
    m_ref[...] = onehot.astype(jnp.float32)


def _select(rrow, srow, kk_real, kkp):
    npad = rrow.shape[1]
    return pl.pallas_call(
        partial(_select_body, kk_real=kk_real, npad=npad),
        grid=(kkp // BP,),
        in_specs=[
            pl.BlockSpec((1, npad), lambda i: (0, 0)),
            pl.BlockSpec((1, npad), lambda i: (0, 0)),
        ],
        out_specs=[
            pl.BlockSpec((BP, 1), lambda i: (i, 0)),
            pl.BlockSpec((BP, 1), lambda i: (i, 0)),
            pl.BlockSpec((BP, npad), lambda i: (i, 0)),
        ],
        out_shape=[
            jax.ShapeDtypeStruct((kkp, 1), jnp.int32),
            jax.ShapeDtypeStruct((kkp, 1), jnp.float32),
            jax.ShapeDtypeStruct((kkp, npad), jnp.float32),
        ],
    )(rrow, srow)


def _sc_gather(idx, tables):
    kkp = idx.shape[0]
    NC, NS = 2, 16
    bpw = kkp // (NC * NS)
    CH = 8
    nt = len(tables)
    mesh = plsc.VectorSubcoreMesh(core_axis_name="c", subcore_axis_name="s")
    outs = [jax.ShapeDtypeStruct((kkp, t.shape[1]), t.dtype) for t in tables]
    scr = ([pltpu.VMEM((CH,), jnp.int32)]
           + [pltpu.VMEM((CH, t.shape[1]), t.dtype) for t in tables]
           + [pltpu.SemaphoreType.DMA])

    @functools.partial(pl.kernel, mesh=mesh, out_type=outs, scratch_types=scr)
    def k(*refs):
        idx_hbm = refs[0]
        tab = refs[1:1 + nt]
        out = refs[1 + nt:1 + 2 * nt]
        idx_v = refs[1 + 2 * nt]
        bufs = refs[2 + 2 * nt:2 + 3 * nt]
        sem = refs[2 + 3 * nt]
        wid = lax.axis_index("s") * NC + lax.axis_index("c")
        base = wid * bpw
        for ch in range(bpw // CH):
            off = base + ch * CH
            pltpu.sync_copy(idx_hbm.at[pl.ds(off, CH)], idx_v)
            for t in range(nt):
                pltpu.async_copy(tab[t].at[idx_v], bufs[t], sem).wait()
                pltpu.sync_copy(bufs[t], out[t].at[pl.ds(off, CH)])

    res = k(idx, *tables)
    return tuple(res) if isinstance(res, (list, tuple)) else (res,)


def _scale_body(c_ref, v_ref, o_ref):
    o_ref[...] = c_ref[...] * v_ref[...]


def _scale(csel, vals):
    kkp, f = csel.shape
    return pl.pallas_call(
        _scale_body,
        grid=(kkp // BP,),
        in_specs=[pl.BlockSpec((BP, f), lambda i: (i, 0)),
                  pl.BlockSpec((BP, 1), lambda i: (i, 0))],
        out_specs=pl.BlockSpec((BP, f), lambda i: (i, 0)),
        out_shape=jax.ShapeDtypeStruct((kkp, f), jnp.float32),
    )(csel, vals)


def _binmm_body(a_ref, b_ref, un_ref, deg_ref, acc_ref, *, nk, kk_real):
    k = pl.program_id(2)
    j = pl.program_id(1)
    m = pl.program_id(0)

    @pl.when(k == 0)
    def _():
        acc_ref[...] = jnp.zeros_like(acc_ref)

    arows = m * BP + jax.lax.broadcasted_iota(jnp.int32, (BP, BP), 0)
    brows = j * BP + jax.lax.broadcasted_iota(jnp.int32, (BP, BP), 0)
    a = ((a_ref[...] != 0) & (arows < kk_real)).astype(jnp.bfloat16)
    b = ((b_ref[...] != 0) & (brows < kk_real)).astype(jnp.bfloat16)
    acc_ref[...] += jax.lax.dot_general(
        a, b, (((1,), (1,)), ((), ())),
        preferred_element_type=jnp.float32)

    @pl.when(k == nk - 1)
    def _():
        un = (acc_ref[...] != 0).astype(jnp.float32)
        un_ref[...] = un
        part = jnp.sum(un, axis=1, keepdims=True)
        deg_ref[...] = jnp.where(j == 0, part, deg_ref[...] + part)


def _binmm(gsel, gtsel, kk_real):
    kkp, npad = gsel.shape
    nk = npad // BP
    return pl.pallas_call(
        partial(_binmm_body, nk=nk, kk_real=kk_real),
        grid=(kkp // BP, kkp // BP, nk),
        in_specs=[
            pl.BlockSpec((BP, BP), lambda m, j, k: (m, k)),
            pl.BlockSpec((BP, BP), lambda m, j, k: (j, k)),
        ],
        out_specs=[
            pl.BlockSpec((BP, BP), lambda m, j, k: (m, j)),
            pl.BlockSpec((BP, 1), lambda m, j, k: (m, 0)),
        ],
        out_shape=[
            jax.ShapeDtypeStruct((kkp, kkp), jnp.float32),
            jax.ShapeDtypeStruct((kkp, 1), jnp.float32),
        ],
        scratch_shapes=[pltpu.VMEM((BP, BP), jnp.float32)],
    )(gsel, gtsel)


def _rownorm_body(un_ref, deg_ref, o_ref):
    d = deg_ref[...]
    o_ref[...] = jnp.where(d > 0, un_ref[...] / d, 0.0)


def _rownorm(un, deg):
    kkp = un.shape[0]
    return pl.pallas_call(
        _rownorm_body,
        grid=(kkp // BP,),
        in_specs=[
            pl.BlockSpec((BP, kkp), lambda i: (i, 0)),
            pl.BlockSpec((BP, 1), lambda i: (i, 0)),
        ],
        out_specs=pl.BlockSpec((BP, kkp), lambda i: (i, 0)),
        out_shape=jax.ShapeDtypeStruct((kkp, kkp), jnp.float32),
    )(un, deg)


def _unpool_body(m_ref, p_ref, base_ref, o_ref):
    acc = jax.lax.dot_general(
        m_ref[...].astype(jnp.bfloat16), p_ref[...].astype(jnp.bfloat16),
        (((0,), (0,)), ((), ())), preferred_element_type=jnp.float32)
    o_ref[...] = base_ref[...] + acc


def _unpool_add(base, p, M):
    nprev, f = base.shape
    kkp = M.shape[0]
    return pl.pallas_call(
        _unpool_body,
        grid=(nprev // BP, f // BP),
        in_specs=[
            pl.BlockSpec((kkp, BP), lambda m, c: (0, m)),
            pl.BlockSpec((kkp, BP), lambda m, c: (0, c)),
            pl.BlockSpec((BP, BP), lambda m, c: (m, c)),
        ],
        out_specs=pl.BlockSpec((BP, BP), lambda m, c: (m, c)),
        out_shape=jax.ShapeDtypeStruct((nprev, f), jnp.float32),
    )(M, p, base)


def _kron_lift(w):
    return jnp.kron(jnp.eye(128 // w.shape[0], dtype=w.dtype), w.T)


def kernel(x, adj, W1, b1, W2, b2, pW1, pb1, pW2, pb2, pW3, pb3, pW4, pb4,
           pW5, pb5):
    b, c, n, l = x.shape
    f = b * c * l

    w1b = [W1[:, i * c:(i + 1) * c] for i in range(GDEP + 1)]
    w2b = [W2[:, i * c:(i + 1) * c] for i in range(GDEP + 1)]
    k1cat = jnp.concatenate([_kron_lift(w) for w in w1b], axis=0)
    k2cat = jnp.concatenate([_kron_lift(w) for w in w2b], axis=0)
    bias1 = jnp.tile(b1, b * l)[None, :]
    bias2 = jnp.tile(b2, b * l)[None, :]
    pws, pbs = [], []
    for pW, pb in ((pW1, pb1), (pW2, pb2), (pW3, pb3), (pW4, pb4), (pW5, pb5)):
        pws.append(pW.reshape(b, c, l).transpose(0, 2, 1).reshape(f, 1))
        pbs.append(pb.reshape(1, 1))

    H = x.transpose(2, 0, 3, 1).reshape(n, f)

    g = adj
    n_real = n
    Xcur = H
    hs, Ms = [], []
    for lvl in range(5):
        npad = _pad_to(n_real)
        if g.shape[0] != npad:
            g = jnp.pad(g, ((0, npad - g.shape[0]), (0, npad - g.shape[1])))
        if Xcur.shape[0] != npad:
            Xcur = jnp.pad(Xcur, ((0, npad - Xcur.shape[0]), (0, 0)))

        gT = _transpose(g)
        eye = jnp.eye(npad, dtype=g.dtype)
        dr = jnp.sum(g + eye, axis=1).reshape(npad, 1)
        dc = jnp.sum(g.T + eye, axis=1).reshape(npad, 1)
        a, at = _normadj(g, gT, dr, dc)

        h1, k1 = _propagate2(a, at, Xcur, Xcur, Xcur)
        h2, k2 = _propagate2(a, at, h1, k1, Xcur)

        C, s = _mix_scores([Xcur, h1, h2, k1, k2], [k1cat, k2cat],
                           [bias1, bias2], pws[lvl], pbs[lvl])

        kk_real = max(2, int(KS[lvl] * n_real))
        kkp = _pad_to(kk_real)
        srow = s.reshape(1, npad)
        rank = _rank(s, srow, n_real)
        idx, vals, M = _select(rank.reshape(1, npad), srow, kk_real, kkp)

        idx_flat = idx.reshape(kkp)
        Craw, gsel, gtsel = _sc_gather(idx_flat, [C, g, gT])
        Csel = _scale(Craw, vals)

        un, deg = _binmm(gsel, gtsel, kk_real)
        g = _rownorm(un, deg)

        hs.append(Csel)
        Ms.append(M)
        Xcur = Csel
        n_real = kk_real

    P = hs[4]
    for lvl in (3, 2, 1, 0):
        P = _unpool_add(hs[lvl], P, Ms[lvl + 1])
    out_flat = _unpool_add(jnp.zeros((n, f), jnp.float32), P, Ms[0])

    return out_flat.reshape(n, b, l, c).transpose(1, 3, 0, 2)
